# baseline (speedup 1.0000x reference)
import functools

import jax
import jax.numpy as jnp
import numpy as np
from jax.experimental import pallas as pl
from jax.experimental.pallas import tpu as pltpu

F32 = jnp.float32
EPS = 1e-5
B, G, NB = 8, 8, 16
HF = 28
S = HF * HF
PH = PW = 5
NPAD = 32
NCOL = G * NPAD
C_OUT = 1536
D_FLAT = C_OUT * PH * PW


def _interp_rows(n_in: int, n_out: int) -> np.ndarray:
    ys = np.linspace(0.0, n_in - 1.0, n_out)
    y0 = np.floor(ys).astype(np.int64)
    y1 = np.minimum(y0 + 1, n_in - 1)
    ly = ys - y0
    r = np.zeros((n_out, n_in), np.float32)
    np.add.at(r, (np.arange(n_out), y0), 1.0 - ly)
    np.add.at(r, (np.arange(n_out), y1), ly)
    return r


def _upsample_mat(n_in: int) -> np.ndarray:
    r = _interp_rows(n_in, HF)
    u = np.einsum('Yh,Xw->hwYX', r, r).reshape(n_in * n_in, S)
    return np.ascontiguousarray(u, np.float32)


_U4T = _upsample_mat(14)
_U5T = _upsample_mat(7)
_E8 = (np.arange(NCOL)[None, :] // NPAD == np.arange(G)[:, None]).astype(np.float32)


def _fused_body(boxes_s, group_s, label_s_ref, c1_s,
                x4_ref, x5_ref, w2_ref, b2_ref, w3_ref, b3_ref,
                u4_ref, u5_ref, skt_ref, e8_ref, n1w_ref, n1b_ref, out_ref):
    b = pl.program_id(0)

    bf16 = jnp.bfloat16
    dn_tb = (((0,), (1,)), ((), ()))
    dn_ta = (((0,), (0,)), ((), ()))
    c4t = jax.lax.dot_general(x4_ref[0], w2_ref[...], dn_tb,
                              preferred_element_type=F32) + b2_ref[...]
    a4 = jax.lax.dot_general(c4t.astype(bf16), u4_ref[...], dn_ta,
                             preferred_element_type=F32)
    c5t = jax.lax.dot_general(x5_ref[0], w3_ref[...], dn_tb,
                              preferred_element_type=F32) + b3_ref[...]
    a5 = jax.lax.dot_general(c5t.astype(bf16), u5_ref[...], dn_ta,
                             preferred_element_type=F32)
    app = jnp.concatenate([a4.astype(bf16), a5.astype(bf16)], axis=0)

    bx1, by1, bx2, by2 = [], [], [], []
    for g in range(G):
        n = b * G + g
        i1 = group_s[2 * n]
        i2 = group_s[2 * n + 1]
        base = b * NB * 4
        bx1.append(jnp.minimum(boxes_s[base + i1 * 4 + 0], boxes_s[base + i2 * 4 + 0]))
        by1.append(jnp.minimum(boxes_s[base + i1 * 4 + 1], boxes_s[base + i2 * 4 + 1]))
        bx2.append(jnp.maximum(boxes_s[base + i1 * 4 + 2], boxes_s[base + i2 * 4 + 2]))
        by2.append(jnp.maximum(boxes_s[base + i1 * 4 + 3], boxes_s[base + i2 * 4 + 3]))

    col = jax.lax.broadcasted_iota(jnp.int32, (1, NCOL), 1)
    cg = col // NPAD
    p = col - cg * NPAD
    x1c = jnp.zeros((1, NCOL), F32)
    y1c = jnp.zeros((1, NCOL), F32)
    x2c = jnp.zeros((1, NCOL), F32)
    y2c = jnp.zeros((1, NCOL), F32)
    for g in range(G):
        mg = cg == g
        x1c = jnp.where(mg, bx1[g], x1c)
        y1c = jnp.where(mg, by1[g], y1c)
        x2c = jnp.where(mg, bx2[g], x2c)
        y2c = jnp.where(mg, by2[g], y2c)
    rw = jnp.maximum(x2c - x1c, 1.0)
    rh = jnp.maximum(y2c - y1c, 1.0)
    pi = (p // PW).astype(F32)
    pj = (p - (p // PW) * PW).astype(F32)
    ysamp = y1c + (pi + 0.5) * (rh * (1.0 / PH))
    xsamp = x1c + (pj + 0.5) * (rw * (1.0 / PW))
    valid = ((ysamp >= -1.0) & (ysamp <= HF) & (xsamp >= -1.0) & (xsamp <= HF)
             & (p < PH * PW))
    vm = jnp.where(valid, 1.0, 0.0)
    y = jnp.clip(ysamp, 0.0, HF - 1.0)
    x = jnp.clip(xsamp, 0.0, HF - 1.0)
    y0f = jnp.floor(y)
    x0f = jnp.floor(x)
    ly = y - y0f
    lx = x - x0f
    y0 = y0f.astype(jnp.int32)
    x0 = x0f.astype(jnp.int32)
    y1 = jnp.minimum(y0 + 1, HF - 1)
    x1 = jnp.minimum(x0 + 1, HF - 1)

    yy = jax.lax.broadcasted_iota(jnp.int32, (HF, HF, NCOL), 0).reshape(S, NCOL)
    xx = jax.lax.broadcasted_iota(jnp.int32, (HF, HF, NCOL), 1).reshape(S, NCOL)
    wyt = (jnp.where(yy == y0, 1.0 - ly, 0.0) + jnp.where(yy == y1, ly, 0.0))
    wxt = (jnp.where(xx == x0, (1.0 - lx) * vm, 0.0)
           + jnp.where(xx == x1, lx * vm, 0.0))

    lab = jnp.zeros((1, G), F32)
    gidx = jax.lax.broadcasted_iota(jnp.int32, (1, G), 1)
    for g in range(G):
        n = b * G + g
        use_g = jnp.where(label_s_ref[n] == -1, 0.0, 1.0)
        lab = jnp.where(gidx == g, use_g, lab)
    sk = skt_ref[0]
    gate = jax.nn.sigmoid(sk * c1_s[0] + c1_s[1])
    geff = gate * lab + (1.0 - lab)
    g8 = jnp.dot(geff, e8_ref[...], preferred_element_type=F32)

    mg_t = wyt * wxt * g8

    roi = jnp.dot(app, mg_t.astype(bf16), preferred_element_type=F32)

    colsum = jnp.sum(roi, axis=0, keepdims=True)
    colsq = jnp.sum(roi * roi, axis=0, keepdims=True)
    dn = (((1,), (1,)), ((), ()))
    s8 = jax.lax.dot_general(colsum, e8_ref[...], dn, preferred_element_type=F32)
    q8 = jax.lax.dot_general(colsq, e8_ref[...], dn, preferred_element_type=F32)
    mu8 = s8 * (1.0 / D_FLAT)
    var8 = q8 * (1.0 / D_FLAT) - mu8 * mu8
    rs8 = jax.lax.rsqrt(var8 + EPS)
    mu_c = jnp.dot(mu8, e8_ref[...], preferred_element_type=F32)
    rs_c = jnp.dot(rs8, e8_ref[...], preferred_element_type=F32)
    v8 = jnp.maximum(
        (roi - mu_c) * rs_c * n1w_ref[...] + n1b_ref[...], 0.0)
    v8h = v8.astype(jnp.bfloat16)
    for g in range(G):
        out_ref[0, g] = v8h[:, g * NPAD:g * NPAD + PH * PW]


def _fc1_body(v_ref, w_ref, b_ref, o_ref):
    k = pl.program_id(1)

    @pl.when(k == 0)
    def _():
        o_ref[...] = jnp.broadcast_to(b_ref[...], o_ref.shape)

    dn = (((1,), (1,)), ((), ()))
    o_ref[...] += jax.lax.dot_general(v_ref[...].astype(F32), w_ref[...], dn,
                                      preferred_element_type=F32)


def _head_body(h_ref, n2w_ref, n2b_ref, w_ref, b_ref, o_ref):
    h = h_ref[...]
    mu = jnp.mean(h, axis=1, keepdims=True)
    xc = h - mu
    var = jnp.mean(xc * xc, axis=1, keepdims=True)
    nh = xc * jax.lax.rsqrt(var + EPS) * n2w_ref[...] + n2b_ref[...]
    dn = (((1,), (1,)), ((), ()))
    o_ref[...] = jax.lax.dot_general(nh, w_ref[...], dn,
                                     preferred_element_type=F32) + b_ref[...]


@functools.partial(jax.jit, static_argnames=())
def kernel(x4, x5, boxes, skeleton, group, real_in_num, label_s,
           conv1_w, conv1_b, conv2_w, conv2_b, conv3_w, conv3_b,
           norm1_w, norm1_b, norm2_w, norm2_b, fc1_w, fc1_b, fc2_w, fc2_b):
    del real_in_num
    x4f = x4.reshape(B, 1024, 196).astype(jnp.bfloat16)
    x5f = x5.reshape(B, 2048, 49).astype(jnp.bfloat16)
    skt = skeleton.reshape(B, G, S).transpose(0, 2, 1).astype(F32)
    boxes_f = boxes.reshape(B * NB * 4).astype(F32)
    group_i = group.reshape(B * G * 2).astype(jnp.int32)
    label_i = label_s.reshape(B * G).astype(jnp.int32)
    c1 = jnp.stack([conv1_w.astype(F32), conv1_b.astype(F32)])
    b2c = conv2_b.reshape(1, 512).astype(F32)
    b3c = conv3_b.reshape(1, 1024).astype(F32)
    w25 = norm1_w.reshape(C_OUT, PH * PW).astype(F32)
    b25 = norm1_b.reshape(C_OUT, PH * PW).astype(F32)
    n1w = jnp.tile(jnp.pad(w25, ((0, 0), (0, NPAD - PH * PW))), (1, G))
    n1b = jnp.tile(jnp.pad(b25, ((0, 0), (0, NPAD - PH * PW))), (1, G))
    u4t = jnp.asarray(_U4T, jnp.bfloat16)
    u5t = jnp.asarray(_U5T, jnp.bfloat16)
    e8 = jnp.asarray(_E8)

    cp = pltpu.CompilerParams(dimension_semantics=("parallel",),
                              vmem_limit_bytes=50 * 1024 * 1024)
    grid_spec = pltpu.PrefetchScalarGridSpec(
        num_scalar_prefetch=4,
        grid=(B,),
        in_specs=[
            pl.BlockSpec((1, 1024, 196), lambda b, *_: (b, 0, 0)),
            pl.BlockSpec((1, 2048, 49), lambda b, *_: (b, 0, 0)),
            pl.BlockSpec((512, 1024), lambda b, *_: (0, 0)),
            pl.BlockSpec((1, 512), lambda b, *_: (0, 0)),
            pl.BlockSpec((1024, 2048), lambda b, *_: (0, 0)),
            pl.BlockSpec((1, 1024), lambda b, *_: (0, 0)),
            pl.BlockSpec((196, S), lambda b, *_: (0, 0)),
            pl.BlockSpec((49, S), lambda b, *_: (0, 0)),
            pl.BlockSpec((1, S, G), lambda b, *_: (b, 0, 0)),
            pl.BlockSpec((G, NCOL), lambda b, *_: (0, 0)),
            pl.BlockSpec((C_OUT, NCOL), lambda b, *_: (0, 0)),
            pl.BlockSpec((C_OUT, NCOL), lambda b, *_: (0, 0)),
        ],
        out_specs=pl.BlockSpec((1, G, C_OUT, PH * PW), lambda b, *_: (b, 0, 0, 0)),
    )
    v8 = pl.pallas_call(
        _fused_body,
        out_shape=jax.ShapeDtypeStruct((B, G, C_OUT, PH * PW), jnp.bfloat16),
        grid_spec=grid_spec,
        compiler_params=cp,
        name="gate_roi_fused",
    )(boxes_f, group_i, label_i, c1,
      x4f, x5f, conv2_w.astype(jnp.bfloat16), b2c, conv3_w.astype(jnp.bfloat16),
      b3c, u4t, u5t, skt, e8, n1w, n1b)

    vflat = v8.reshape(B * G, D_FLAT)

    kb = 6400
    nk = D_FLAT // kb
    h = pl.pallas_call(
        _fc1_body,
        out_shape=jax.ShapeDtypeStruct((B * G, 512), F32),
        grid=(2, nk),
        in_specs=[
            pl.BlockSpec((B * G, kb), lambda j, k: (0, k)),
            pl.BlockSpec((256, kb), lambda j, k: (j, k)),
            pl.BlockSpec((1, 256), lambda j, k: (0, j)),
        ],
        out_specs=pl.BlockSpec((B * G, 256), lambda j, k: (0, j)),
        compiler_params=pltpu.CompilerParams(
            dimension_semantics=("parallel", "arbitrary"),
            vmem_limit_bytes=50 * 1024 * 1024),
        name="fc1",
    )(vflat, fc1_w.astype(F32), fc1_b.reshape(1, 512).astype(F32))

    out = pl.pallas_call(
        _head_body,
        out_shape=jax.ShapeDtypeStruct((B * G, 6), F32),
        name="ln2_fc2",
    )(h, norm2_w.reshape(1, 512).astype(F32), norm2_b.reshape(1, 512).astype(F32),
      fc2_w.astype(F32), fc2_b.reshape(1, 6).astype(F32))
    return out


# speedup vs baseline: 1.1189x; 1.1189x over previous
import functools

import jax
import jax.numpy as jnp
import numpy as np
from jax.experimental import pallas as pl
from jax.experimental.pallas import tpu as pltpu

F32 = jnp.float32
EPS = 1e-5
B, G, NB = 8, 8, 16
HF = 28
S = HF * HF
PH = PW = 5
NPAD = 32
NCOL = G * NPAD
C_OUT = 1536
D_FLAT = C_OUT * PH * PW


def _interp_rows(n_in: int, n_out: int) -> np.ndarray:
    ys = np.linspace(0.0, n_in - 1.0, n_out)
    y0 = np.floor(ys).astype(np.int64)
    y1 = np.minimum(y0 + 1, n_in - 1)
    ly = ys - y0
    r = np.zeros((n_out, n_in), np.float32)
    np.add.at(r, (np.arange(n_out), y0), 1.0 - ly)
    np.add.at(r, (np.arange(n_out), y1), ly)
    return r


def _upsample_mat(n_in: int) -> np.ndarray:
    r = _interp_rows(n_in, HF)
    u = np.einsum('Yh,Xw->hwYX', r, r).reshape(n_in * n_in, S)
    return np.ascontiguousarray(u, np.float32)


_U4T = _upsample_mat(14)
_U5T = _upsample_mat(7)
_E8 = (np.arange(NCOL)[None, :] // NPAD == np.arange(G)[:, None]).astype(np.float32)


def _fused_body(boxes_s, group_s, label_s_ref, c1_s,
                x4_ref, x5_ref, w2_ref, b2_ref, w3_ref, b3_ref,
                u4_ref, u5_ref, skt_ref, e8_ref, out_ref):
    b = pl.program_id(0)

    dn_tb = (((0,), (1,)), ((), ()))
    dn_ta = (((0,), (0,)), ((), ()))
    c4t = jax.lax.dot_general(x4_ref[0], w2_ref[...], dn_tb,
                              preferred_element_type=F32) + b2_ref[...]
    a4 = jax.lax.dot_general(c4t, u4_ref[...], dn_ta,
                             preferred_element_type=F32)
    c5t = jax.lax.dot_general(x5_ref[0], w3_ref[...], dn_tb,
                              preferred_element_type=F32) + b3_ref[...]
    a5 = jax.lax.dot_general(c5t, u5_ref[...], dn_ta,
                             preferred_element_type=F32)
    app = jnp.concatenate([a4, a5], axis=0)

    bx1, by1, bx2, by2 = [], [], [], []
    for g in range(G):
        n = b * G + g
        i1 = group_s[2 * n]
        i2 = group_s[2 * n + 1]
        base = b * NB * 4
        bx1.append(jnp.minimum(boxes_s[base + i1 * 4 + 0], boxes_s[base + i2 * 4 + 0]))
        by1.append(jnp.minimum(boxes_s[base + i1 * 4 + 1], boxes_s[base + i2 * 4 + 1]))
        bx2.append(jnp.maximum(boxes_s[base + i1 * 4 + 2], boxes_s[base + i2 * 4 + 2]))
        by2.append(jnp.maximum(boxes_s[base + i1 * 4 + 3], boxes_s[base + i2 * 4 + 3]))

    col = jax.lax.broadcasted_iota(jnp.int32, (1, NCOL), 1)
    cg = col // NPAD
    p = col - cg * NPAD
    x1c = jnp.zeros((1, NCOL), F32)
    y1c = jnp.zeros((1, NCOL), F32)
    x2c = jnp.zeros((1, NCOL), F32)
    y2c = jnp.zeros((1, NCOL), F32)
    for g in range(G):
        mg = cg == g
        x1c = jnp.where(mg, bx1[g], x1c)
        y1c = jnp.where(mg, by1[g], y1c)
        x2c = jnp.where(mg, bx2[g], x2c)
        y2c = jnp.where(mg, by2[g], y2c)
    rw = jnp.maximum(x2c - x1c, 1.0)
    rh = jnp.maximum(y2c - y1c, 1.0)
    pi = (p // PW).astype(F32)
    pj = (p - (p // PW) * PW).astype(F32)
    ysamp = y1c + (pi + 0.5) * (rh * (1.0 / PH))
    xsamp = x1c + (pj + 0.5) * (rw * (1.0 / PW))
    valid = ((ysamp >= -1.0) & (ysamp <= HF) & (xsamp >= -1.0) & (xsamp <= HF)
             & (p < PH * PW))
    vm = jnp.where(valid, 1.0, 0.0)
    y = jnp.clip(ysamp, 0.0, HF - 1.0)
    x = jnp.clip(xsamp, 0.0, HF - 1.0)
    y0f = jnp.floor(y)
    x0f = jnp.floor(x)
    ly = y - y0f
    lx = x - x0f
    y0 = y0f.astype(jnp.int32)
    x0 = x0f.astype(jnp.int32)
    y1 = jnp.minimum(y0 + 1, HF - 1)
    x1 = jnp.minimum(x0 + 1, HF - 1)

    yy = jax.lax.broadcasted_iota(jnp.int32, (HF, HF, NCOL), 0).reshape(S, NCOL)
    xx = jax.lax.broadcasted_iota(jnp.int32, (HF, HF, NCOL), 1).reshape(S, NCOL)
    wyt = (jnp.where(yy == y0, 1.0 - ly, 0.0) + jnp.where(yy == y1, ly, 0.0))
    wxt = (jnp.where(xx == x0, (1.0 - lx) * vm, 0.0)
           + jnp.where(xx == x1, lx * vm, 0.0))

    lab = jnp.zeros((1, G), F32)
    gidx = jax.lax.broadcasted_iota(jnp.int32, (1, G), 1)
    for g in range(G):
        n = b * G + g
        use_g = jnp.where(label_s_ref[n] == -1, 0.0, 1.0)
        lab = jnp.where(gidx == g, use_g, lab)
    sk = skt_ref[0]
    gate = jax.nn.sigmoid(sk * c1_s[0] + c1_s[1])
    geff = gate * lab + (1.0 - lab)
    g8 = jnp.dot(geff, e8_ref[...], preferred_element_type=F32)

    mg_t = wyt * wxt * g8

    roi = jnp.dot(app, mg_t, preferred_element_type=F32)

    colsum = jnp.sum(roi, axis=0, keepdims=True)
    colsq = jnp.sum(roi * roi, axis=0, keepdims=True)
    dn = (((1,), (1,)), ((), ()))
    s8 = jax.lax.dot_general(colsum, e8_ref[...], dn, preferred_element_type=F32)
    q8 = jax.lax.dot_general(colsq, e8_ref[...], dn, preferred_element_type=F32)
    mu8 = s8 * (1.0 / D_FLAT)
    var8 = q8 * (1.0 / D_FLAT) - mu8 * mu8
    rs8 = jax.lax.rsqrt(var8 + EPS)
    mu_c = jnp.dot(mu8, e8_ref[...], preferred_element_type=F32)
    rs_c = jnp.dot(rs8, e8_ref[...], preferred_element_type=F32)
    v8h = ((roi - mu_c) * rs_c).astype(jnp.bfloat16)
    for g in range(G):
        out_ref[0, g] = v8h[:, g * NPAD:g * NPAD + PH * PW]


def _fc1_head_body(v_ref, w_ref, n1w_ref, n1b_ref, b1_ref,
                   n2w_ref, n2b_ref, w2_ref, b2_ref, o_ref, acc_ref, nk):
    k = pl.program_id(0)

    @pl.when(k == 0)
    def _():
        acc_ref[...] = jnp.broadcast_to(b1_ref[...], acc_ref.shape)

    lhs = jnp.maximum(
        v_ref[...].astype(F32) * n1w_ref[...] + n1b_ref[...], 0.0)
    dn = (((1,), (1,)), ((), ()))
    acc_ref[...] += jax.lax.dot_general(lhs, w_ref[...], dn,
                                        preferred_element_type=F32)

    @pl.when(k == nk - 1)
    def _():
        h = acc_ref[...]
        mu = jnp.mean(h, axis=1, keepdims=True)
        xc = h - mu
        var = jnp.mean(xc * xc, axis=1, keepdims=True)
        nh = xc * jax.lax.rsqrt(var + EPS) * n2w_ref[...] + n2b_ref[...]
        o_ref[...] = jax.lax.dot_general(nh, w2_ref[...], dn,
                                         preferred_element_type=F32) + b2_ref[...]


@functools.partial(jax.jit, static_argnames=())
def kernel(x4, x5, boxes, skeleton, group, real_in_num, label_s,
           conv1_w, conv1_b, conv2_w, conv2_b, conv3_w, conv3_b,
           norm1_w, norm1_b, norm2_w, norm2_b, fc1_w, fc1_b, fc2_w, fc2_b):
    del real_in_num
    x4f = x4.reshape(B, 1024, 196).astype(F32)
    x5f = x5.reshape(B, 2048, 49).astype(F32)
    skt = skeleton.reshape(B, G, S).transpose(0, 2, 1).astype(F32)
    boxes_f = boxes.reshape(B * NB * 4).astype(F32)
    group_i = group.reshape(B * G * 2).astype(jnp.int32)
    label_i = label_s.reshape(B * G).astype(jnp.int32)
    c1 = jnp.stack([conv1_w.astype(F32), conv1_b.astype(F32)])
    b2c = conv2_b.reshape(1, 512).astype(F32)
    b3c = conv3_b.reshape(1, 1024).astype(F32)
    u4t = jnp.asarray(_U4T)
    u5t = jnp.asarray(_U5T)
    e8 = jnp.asarray(_E8)

    cp = pltpu.CompilerParams(dimension_semantics=("parallel",),
                              vmem_limit_bytes=50 * 1024 * 1024)
    grid_spec = pltpu.PrefetchScalarGridSpec(
        num_scalar_prefetch=4,
        grid=(B,),
        in_specs=[
            pl.BlockSpec((1, 1024, 196), lambda b, *_: (b, 0, 0)),
            pl.BlockSpec((1, 2048, 49), lambda b, *_: (b, 0, 0)),
            pl.BlockSpec((512, 1024), lambda b, *_: (0, 0)),
            pl.BlockSpec((1, 512), lambda b, *_: (0, 0)),
            pl.BlockSpec((1024, 2048), lambda b, *_: (0, 0)),
            pl.BlockSpec((1, 1024), lambda b, *_: (0, 0)),
            pl.BlockSpec((196, S), lambda b, *_: (0, 0)),
            pl.BlockSpec((49, S), lambda b, *_: (0, 0)),
            pl.BlockSpec((1, S, G), lambda b, *_: (b, 0, 0)),
            pl.BlockSpec((G, NCOL), lambda b, *_: (0, 0)),
        ],
        out_specs=pl.BlockSpec((1, G, C_OUT, PH * PW), lambda b, *_: (b, 0, 0, 0)),
    )
    v8 = pl.pallas_call(
        _fused_body,
        out_shape=jax.ShapeDtypeStruct((B, G, C_OUT, PH * PW), jnp.bfloat16),
        grid_spec=grid_spec,
        compiler_params=cp,
        name="gate_roi_fused",
    )(boxes_f, group_i, label_i, c1,
      x4f, x5f, conv2_w.astype(F32), b2c, conv3_w.astype(F32), b3c,
      u4t, u5t, skt, e8)

    vflat = v8.reshape(B * G, D_FLAT)

    kb = 6400
    nk = D_FLAT // kb
    out = pl.pallas_call(
        functools.partial(_fc1_head_body, nk=nk),
        out_shape=jax.ShapeDtypeStruct((B * G, 6), F32),
        grid=(nk,),
        in_specs=[
            pl.BlockSpec((B * G, kb), lambda k: (0, k)),
            pl.BlockSpec((512, kb), lambda k: (0, k)),
            pl.BlockSpec((1, kb), lambda k: (0, k)),
            pl.BlockSpec((1, kb), lambda k: (0, k)),
            pl.BlockSpec((1, 512), lambda k: (0, 0)),
            pl.BlockSpec((1, 512), lambda k: (0, 0)),
            pl.BlockSpec((1, 512), lambda k: (0, 0)),
            pl.BlockSpec((6, 512), lambda k: (0, 0)),
            pl.BlockSpec((1, 6), lambda k: (0, 0)),
        ],
        out_specs=pl.BlockSpec((B * G, 6), lambda k: (0, 0)),
        scratch_shapes=[pltpu.VMEM((B * G, 512), F32)],
        compiler_params=pltpu.CompilerParams(
            dimension_semantics=("arbitrary",),
            vmem_limit_bytes=50 * 1024 * 1024),
        name="fc1_head",
    )(vflat, fc1_w.astype(F32),
      norm1_w.reshape(1, D_FLAT).astype(F32),
      norm1_b.reshape(1, D_FLAT).astype(F32),
      fc1_b.reshape(1, 512).astype(F32),
      norm2_w.reshape(1, 512).astype(F32), norm2_b.reshape(1, 512).astype(F32),
      fc2_w.astype(F32), fc2_b.reshape(1, 6).astype(F32))
    return out
